# Initial kernel scaffold; baseline (speedup 1.0000x reference)
import jax
import jax.numpy as jnp
from jax.experimental import pallas as pl


def kernel(x, pos, edge_index, edge_attr, batch, lin_in_W, lin_in_b, msg_W1, msg_b1, msg_g, msg_beta, msg_W2, msg_b2, upd_W1, upd_b1, upd_g, upd_beta, upd_W2, upd_b2, pred_W, pred_b):
    raise NotImplementedError("write your pallas kernel here")


# speedup vs baseline: 1.2425x; 1.2425x over previous
import functools

import jax
import jax.numpy as jnp
from jax import lax
from jax.experimental import pallas as pl
from jax.experimental.pallas import tpu as pltpu
from jax.experimental.pallas import tpu_sc as plsc

N = 50000
E = 800000
D = 64
EDGE_D = 4
L = 4
G = 64
EPS = 1e-5

NC = 2
NS = 16
NW = NC * NS

EPW = E // NW
P1_BE = 128
P1_NB = EPW // P1_BE
P1_TAIL = EPW - P1_NB * P1_BE

EPS_SC = E // NS
SC_BE = 128
SC_NB = EPS_SC // SC_BE
SC_TAIL = EPS_SC - SC_NB * SC_BE
DH = D // NC
NPS = 3136
NPAD = NPS * NS
ACC_R = NPAD + 8
ZCH = 16

NBLK = 5000
N_NB = N // NBLK
MBLK = 4000
M_NB = E // MBLK

_mesh = plsc.VectorSubcoreMesh(core_axis_name="c", subcore_axis_name="s")


def _msg_stats_body(ab_hbm, dst_hbm, src_hbm, ea_hbm, w1c_hbm,
                    m_hbm, stats_hbm,
                    didx, sidx, didx_t, sidx_t, a_v, b_v, ea_v, m_v,
                    w1c_v, sacc, stats_v, sem):
    c = lax.axis_index("c")
    s = lax.axis_index("s")
    wid = s * NC + c
    base0 = wid * EPW
    pltpu.sync_copy(w1c_hbm, w1c_v)
    for g in range(8):
        sacc[g] = jnp.zeros((16,), jnp.float32)
    w1c_vecs = [[w1c_v[k, pl.ds(g * 16, 16)] for g in range(4)]
                for k in range(4)]

    def do_block(di_ref, si_ref, base, ne):
        pltpu.sync_copy(dst_hbm.at[pl.ds(base, ne)], di_ref)
        pltpu.sync_copy(src_hbm.at[pl.ds(base, ne)], si_ref)
        pltpu.async_copy(ab_hbm.at[di_ref], a_v.at[pl.ds(0, ne)], sem).wait()
        pltpu.async_copy(ab_hbm.at[si_ref], b_v.at[pl.ds(0, ne)], sem).wait()
        pltpu.sync_copy(ea_hbm.at[pl.ds(base * EDGE_D, ne * EDGE_D)],
                        ea_v.at[pl.ds(0, ne * EDGE_D)])

        def edge_body(e, _):
            ve = ea_v[pl.ds(e * EDGE_D, 16)]
            for g in range(4):
                v = (a_v[e, pl.ds(g * 16, 16)]
                     + b_v[e, pl.ds(D + g * 16, 16)])
                for k in range(4):
                    v = v + ve[k] * w1c_vecs[k][g]
                v = jnp.maximum(v, 0.0)
                m_v[e, pl.ds(g * 16, 16)] = v
                sacc[g] = sacc[g] + v
                sacc[g + 4] = sacc[g + 4] + v * v
            return 0

        lax.fori_loop(0, ne, edge_body, 0)
        pltpu.sync_copy(m_v.at[pl.ds(0, ne)], m_hbm.at[pl.ds(base, ne)])

    def blk_body(i, _):
        do_block(didx, sidx, base0 + i * P1_BE, P1_BE)
        return 0

    lax.fori_loop(0, P1_NB, blk_body, 0)
    do_block(didx_t, sidx_t, base0 + P1_NB * P1_BE, P1_TAIL)

    for g in range(4):
        stats_v[0, pl.ds(g * 16, 16)] = sacc[g]
        stats_v[1, pl.ds(g * 16, 16)] = sacc[g + 4]
    pltpu.sync_copy(stats_v, stats_hbm.at[wid])


_msg_stats = pl.kernel(
    _msg_stats_body,
    out_type=[jax.ShapeDtypeStruct((E, D), jnp.float32),
              jax.ShapeDtypeStruct((NW, 2, D), jnp.float32)],
    mesh=_mesh,
    scratch_types=[pltpu.VMEM((P1_BE,), jnp.int32),
                   pltpu.VMEM((P1_BE,), jnp.int32),
                   pltpu.VMEM((P1_TAIL,), jnp.int32),
                   pltpu.VMEM((P1_TAIL,), jnp.int32),
                   pltpu.VMEM((P1_BE, 2 * D), jnp.float32),
                   pltpu.VMEM((P1_BE, 2 * D), jnp.float32),
                   pltpu.VMEM((P1_BE * EDGE_D + 16,), jnp.float32),
                   pltpu.VMEM((P1_BE, D), jnp.float32),
                   pltpu.VMEM((EDGE_D, D), jnp.float32),
                   pltpu.VMEM((8, 16), jnp.float32),
                   pltpu.VMEM((2, D), jnp.float32),
                   pltpu.SemaphoreType.DMA],
    name="msg_stats",
)


def _scatter_body(m2_hbm, dst_hbm, aggr_hbm,
                  didx, didx_t, cidx, mb_v, zb_v, st_v, acc):
    c = lax.axis_index("c")
    s = lax.axis_index("s")

    def zb_body(i, _):
        zb_v[i, pl.ds(0, 16)] = jnp.zeros((16,), jnp.float32)
        zb_v[i, pl.ds(16, 16)] = jnp.zeros((16,), jnp.float32)
        return 0

    lax.fori_loop(0, ZCH, zb_body, 0)
    lanes = lax.iota(jnp.int32, 16)

    def zcp_body(j, _):
        cidx[...] = s * NPS + j * ZCH + lanes
        pltpu.sync_copy(zb_v, acc.at[cidx])
        return 0

    lax.fori_loop(0, NPS // ZCH, zcp_body, 0)
    plsc.subcore_barrier()

    base0 = s * EPS_SC

    def do_blk(idx_ref, base, ne):
        pltpu.sync_copy(dst_hbm.at[pl.ds(base, ne)], idx_ref)
        pltpu.sync_copy(m2_hbm.at[c, pl.ds(base, ne)], mb_v.at[pl.ds(0, ne)])
        pltpu.sync_copy(mb_v.at[pl.ds(0, ne)], acc.at[idx_ref], add=True)

    def blk_body(i, _):
        do_blk(didx, base0 + i * SC_BE, SC_BE)
        return 0

    lax.fori_loop(0, SC_NB, blk_body, 0)
    do_blk(didx_t, base0 + SC_NB * SC_BE, SC_TAIL)
    plsc.subcore_barrier()

    def wcp_body(j, _):
        cidx[...] = s * NPS + j * ZCH + lanes
        pltpu.sync_copy(acc.at[cidx], st_v)
        pltpu.sync_copy(st_v, aggr_hbm.at[c, pl.ds(s * NPS + j * ZCH, ZCH)])
        return 0

    lax.fori_loop(0, NPS // ZCH, wcp_body, 0)


_scatter = pl.kernel(
    _scatter_body,
    out_type=jax.ShapeDtypeStruct((NC, NPAD, DH), jnp.float32),
    mesh=_mesh,
    scratch_types=[pltpu.VMEM((SC_BE,), jnp.int32),
                   pltpu.VMEM((SC_TAIL,), jnp.int32),
                   pltpu.VMEM((ZCH,), jnp.int32),
                   pltpu.VMEM((SC_BE, DH), jnp.float32),
                   pltpu.VMEM((ZCH, DH), jnp.float32),
                   pltpu.VMEM((ZCH, DH), jnp.float32),
                   pltpu.VMEM_SHARED((ACC_R, DH), jnp.float32)],
    compiler_params=pltpu.CompilerParams(use_tc_tiling_on_sc=False),
    name="edge_scatter",
)


def _dot(a, b):
    return jnp.dot(a, b, preferred_element_type=jnp.float32)


def _embed_body(xp_ref, w_ref, b_ref, wa_ref, wb_ref, b1_ref,
                h_ref, ab_ref):
    h = _dot(xp_ref[...], w_ref[...]) + b_ref[...]
    h_ref[...] = h
    ab_ref[...] = jnp.concatenate(
        [_dot(h, wa_ref[...]) + b1_ref[...], _dot(h, wb_ref[...])], axis=1)


def _embed_call(xp, w, b, wa, wb, b1):
    row = lambda i: (i, 0)
    return pl.pallas_call(
        _embed_body,
        grid=(N_NB,),
        in_specs=[pl.BlockSpec((NBLK, ATOM3), row),
                  pl.BlockSpec((ATOM3, D), lambda i: (0, 0)),
                  pl.BlockSpec((1, D), lambda i: (0, 0)),
                  pl.BlockSpec((D, D), lambda i: (0, 0)),
                  pl.BlockSpec((D, D), lambda i: (0, 0)),
                  pl.BlockSpec((1, D), lambda i: (0, 0))],
        out_specs=[pl.BlockSpec((NBLK, D), row),
                   pl.BlockSpec((NBLK, 2 * D), row)],
        out_shape=[jax.ShapeDtypeStruct((N, D), jnp.float32),
                   jax.ShapeDtypeStruct((N, 2 * D), jnp.float32)],
    )(xp, w, b, wa, wb, b1)


ATOM3 = 14


def _mm_body(m_ref, w_ref, b_ref, o_ref):
    m2 = jnp.maximum(_dot(m_ref[...], w_ref[...]) + b_ref[...], 0.0)
    o_ref[0] = m2[:, :DH]
    o_ref[1] = m2[:, DH:]


def _mm_call(m, w2eff, b2eff):
    return pl.pallas_call(
        _mm_body,
        grid=(M_NB,),
        in_specs=[pl.BlockSpec((MBLK, D), lambda i: (i, 0)),
                  pl.BlockSpec((D, D), lambda i: (0, 0)),
                  pl.BlockSpec((1, D), lambda i: (0, 0))],
        out_specs=pl.BlockSpec((NC, MBLK, DH), lambda i: (0, i, 0)),
        out_shape=jax.ShapeDtypeStruct((NC, E, DH), jnp.float32),
    )(m, w2eff, b2eff)


def _upd1_body(h_ref, ag_ref, ua_ref, ub_ref, b1_ref, u1_ref, st_ref):
    i = pl.program_id(0)
    ag = jnp.concatenate([ag_ref[0], ag_ref[1]], axis=1)
    u1 = jnp.maximum(_dot(h_ref[...], ua_ref[...])
                     + _dot(ag, ub_ref[...]) + b1_ref[...], 0.0)
    u1_ref[...] = u1
    ps = jnp.sum(u1, axis=0, keepdims=True)
    pq = jnp.sum(u1 * u1, axis=0, keepdims=True)
    blk = jnp.concatenate([ps, pq], axis=0)

    @pl.when(i == 0)
    def _():
        st_ref[...] = blk

    @pl.when(i > 0)
    def _():
        st_ref[...] = st_ref[...] + blk


def _upd1_call(h, aggr, ua, ub, b1):
    return pl.pallas_call(
        _upd1_body,
        grid=(N_NB,),
        in_specs=[pl.BlockSpec((NBLK, D), lambda i: (i, 0)),
                  pl.BlockSpec((NC, NBLK, DH), lambda i: (0, i, 0)),
                  pl.BlockSpec((D, D), lambda i: (0, 0)),
                  pl.BlockSpec((D, D), lambda i: (0, 0)),
                  pl.BlockSpec((1, D), lambda i: (0, 0))],
        out_specs=[pl.BlockSpec((NBLK, D), lambda i: (i, 0)),
                   pl.BlockSpec((2, D), lambda i: (0, 0))],
        out_shape=[jax.ShapeDtypeStruct((N, D), jnp.float32),
                   jax.ShapeDtypeStruct((2, D), jnp.float32)],
    )(h, aggr, ua, ub, b1)


def _upd2ab_body(h_ref, u1_ref, u2_ref, b2_ref, wa_ref, wb_ref, b1n_ref,
                 hn_ref, ab_ref):
    u = jnp.maximum(_dot(u1_ref[...], u2_ref[...]) + b2_ref[...], 0.0)
    hn = h_ref[...] + u
    hn_ref[...] = hn
    ab_ref[...] = jnp.concatenate(
        [_dot(hn, wa_ref[...]) + b1n_ref[...], _dot(hn, wb_ref[...])], axis=1)


def _upd2ab_call(h, u1, u2eff, ub2eff, wa, wb, b1n):
    return pl.pallas_call(
        _upd2ab_body,
        grid=(N_NB,),
        in_specs=[pl.BlockSpec((NBLK, D), lambda i: (i, 0)),
                  pl.BlockSpec((NBLK, D), lambda i: (i, 0)),
                  pl.BlockSpec((D, D), lambda i: (0, 0)),
                  pl.BlockSpec((1, D), lambda i: (0, 0)),
                  pl.BlockSpec((D, D), lambda i: (0, 0)),
                  pl.BlockSpec((D, D), lambda i: (0, 0)),
                  pl.BlockSpec((1, D), lambda i: (0, 0))],
        out_specs=[pl.BlockSpec((NBLK, D), lambda i: (i, 0)),
                   pl.BlockSpec((NBLK, 2 * D), lambda i: (i, 0))],
        out_shape=[jax.ShapeDtypeStruct((N, D), jnp.float32),
                   jax.ShapeDtypeStruct((N, 2 * D), jnp.float32)],
    )(h, u1, u2eff, ub2eff, wa, wb, b1n)


def _upd2_body(h_ref, u1_ref, u2_ref, b2_ref, hn_ref):
    u = jnp.maximum(_dot(u1_ref[...], u2_ref[...]) + b2_ref[...], 0.0)
    hn_ref[...] = h_ref[...] + u


def _upd2_call(h, u1, u2eff, ub2eff):
    return pl.pallas_call(
        _upd2_body,
        grid=(N_NB,),
        in_specs=[pl.BlockSpec((NBLK, D), lambda i: (i, 0)),
                  pl.BlockSpec((NBLK, D), lambda i: (i, 0)),
                  pl.BlockSpec((D, D), lambda i: (0, 0)),
                  pl.BlockSpec((1, D), lambda i: (0, 0))],
        out_specs=pl.BlockSpec((NBLK, D), lambda i: (i, 0)),
        out_shape=jax.ShapeDtypeStruct((N, D), jnp.float32),
    )(h, u1, u2eff, ub2eff)


def _pool_body(b_ref, h_ref, pw_ref, pb_ref, out_ref, sums, cnts):
    i = pl.program_id(0)

    @pl.when(i == 0)
    def _():
        sums[...] = jnp.zeros_like(sums)
        cnts[...] = jnp.zeros_like(cnts)

    brow = b_ref[0]
    gids = lax.broadcasted_iota(jnp.int32, (G, NBLK), 0)
    onehot = (gids == brow).astype(jnp.float32)
    sums[...] += _dot(onehot, h_ref[...])
    cnts[...] += jnp.sum(onehot, axis=1, keepdims=True)

    @pl.when(i == N_NB - 1)
    def _():
        hg = sums[...] / jnp.maximum(cnts[...], 1.0)
        out_ref[...] = _dot(hg, pw_ref[...]) + pb_ref[...]


def _pool_call(batch3, h, pw, pb):
    return pl.pallas_call(
        _pool_body,
        grid=(N_NB,),
        in_specs=[pl.BlockSpec((1, 1, NBLK), lambda i: (i, 0, 0)),
                  pl.BlockSpec((NBLK, D), lambda i: (i, 0)),
                  pl.BlockSpec((D, 1), lambda i: (0, 0)),
                  pl.BlockSpec((1, 1), lambda i: (0, 0))],
        out_specs=pl.BlockSpec((G, 1), lambda i: (0, 0)),
        out_shape=jax.ShapeDtypeStruct((G, 1), jnp.float32),
        scratch_shapes=[pltpu.VMEM((G, D), jnp.float32),
                        pltpu.VMEM((G, 1), jnp.float32)],
    )(batch3, h, pw, pb)


def kernel(x, pos, edge_index, edge_attr, batch, lin_in_W, lin_in_b,
           msg_W1, msg_b1, msg_g, msg_beta, msg_W2, msg_b2,
           upd_W1, upd_b1, upd_g, upd_beta, upd_W2, upd_b2,
           pred_W, pred_b):
    src = edge_index[0]
    dst = edge_index[1]
    eaf = edge_attr.reshape(E * EDGE_D)
    xp = jnp.concatenate([x, pos], axis=1)
    h, AB = _embed_call(xp, lin_in_W, lin_in_b.reshape(1, D),
                        msg_W1[0, :D], msg_W1[0, D:2 * D],
                        msg_b1[0].reshape(1, D))
    for l in range(L):
        w1c = msg_W1[l, 2 * D:]
        m, pstats = _msg_stats(AB, dst, src, eaf, w1c)
        st = jnp.sum(pstats, axis=0)
        mu = st[0] / E
        var = st[1] / E - mu * mu
        sg = msg_g[l] * lax.rsqrt(var + EPS)
        t = msg_beta[l] - mu * sg
        w2eff = sg[:, None] * msg_W2[l]
        b2eff = t @ msg_W2[l] + msg_b2[l]
        m2 = _mm_call(m, w2eff, b2eff.reshape(1, D))
        aggr = _scatter(m2, dst)
        u1, st2 = _upd1_call(h, aggr, upd_W1[l, :D], upd_W1[l, D:],
                             upd_b1[l].reshape(1, D))
        mu2 = st2[0] / N
        var2 = st2[1] / N - mu2 * mu2
        sg2 = upd_g[l] * lax.rsqrt(var2 + EPS)
        t2 = upd_beta[l] - mu2 * sg2
        u2eff = sg2[:, None] * upd_W2[l]
        ub2eff = t2 @ upd_W2[l] + upd_b2[l]
        if l < L - 1:
            h, AB = _upd2ab_call(h, u1, u2eff, ub2eff.reshape(1, D),
                                 msg_W1[l + 1, :D], msg_W1[l + 1, D:2 * D],
                                 msg_b1[l + 1].reshape(1, D))
        else:
            h = _upd2_call(h, u1, u2eff, ub2eff.reshape(1, D))
    out = _pool_call(batch.reshape(N_NB, 1, NBLK), h, pred_W,
                     pred_b.reshape(1, 1))
    return out.reshape(-1)


# speedup vs baseline: 1.5016x; 1.2086x over previous
import functools

import jax
import jax.numpy as jnp
from jax import lax
from jax.experimental import pallas as pl
from jax.experimental.pallas import tpu as pltpu
from jax.experimental.pallas import tpu_sc as plsc

N = 50000
E = 800000
D = 64
EDGE_D = 4
L = 4
G = 64
EPS = 1e-5

NC = 2
NS = 16
NW = NC * NS

EPW = E // NW
P1_BE = 128
SB = 2048
NSB = 12
SB_NB = SB // P1_BE
P1_TB = EPW - NSB * SB
P1_TNB = P1_TB // P1_BE
P1_TAIL = P1_TB - P1_TNB * P1_BE

EPS_SC = E // NS
SC_BE = 128
SC_NB = EPS_SC // SC_BE
SC_TAIL = EPS_SC - SC_NB * SC_BE
DH = D // NC
NPS = 3136
NPAD = NPS * NS
ACC_R = NPAD + 8
ZCH = 16

NBLK = 5000
N_NB = N // NBLK
MBLK = 4000
M_NB = E // MBLK

_mesh = plsc.VectorSubcoreMesh(core_axis_name="c", subcore_axis_name="s")


def _msg_stats_body(ab_hbm, dst_hbm, src_hbm, ea_hbm, w1c_hbm,
                    m_hbm, stats_hbm,
                    dsb0, dsb1, ssb0, ssb1, esb0, esb1, a_v, b_v, m_v,
                    w1c_v, sacc, stats_v,
                    sem_i0, sem_i1, sem_e0, sem_e1,
                    sem_a0, sem_a1, sem_b0, sem_b1):
    c = lax.axis_index("c")
    s = lax.axis_index("s")
    wid = s * NC + c
    base0 = wid * EPW
    dsb = [dsb0, dsb1]
    ssb = [ssb0, ssb1]
    esb = [esb0, esb1]
    sems_i = [sem_i0, sem_i1]
    sems_e = [sem_e0, sem_e1]
    sems_a = [sem_a0, sem_a1]
    sems_b = [sem_b0, sem_b1]
    pltpu.sync_copy(w1c_hbm, w1c_v)
    for g in range(8):
        sacc[g] = jnp.zeros((16,), jnp.float32)
    w1c_vecs = [[w1c_v[k, pl.ds(g * 16, 16)] for g in range(4)]
                for k in range(4)]

    def start_sb(jj, par):
        b = base0 + jj * SB
        pltpu.async_copy(dst_hbm.at[pl.ds(b, SB)], dsb[par], sems_i[par])
        pltpu.async_copy(src_hbm.at[pl.ds(b, SB)], ssb[par], sems_i[par])
        pltpu.async_copy(ea_hbm.at[pl.ds(b * EDGE_D, SB * EDGE_D)],
                         esb[par].at[pl.ds(0, SB * EDGE_D)], sems_e[par])

    def start_gather(par, k, gs):
        pltpu.async_copy(ab_hbm.at[dsb[par].at[pl.ds(k * P1_BE, P1_BE)]],
                         a_v.at[gs], sems_a[gs])
        pltpu.async_copy(ab_hbm.at[ssb[par].at[pl.ds(k * P1_BE, P1_BE)]],
                         b_v.at[gs], sems_b[gs])

    def wait_gather(gs):
        pltpu.make_async_copy(ab_hbm.at[pl.ds(0, P1_BE)], a_v.at[gs],
                              sems_a[gs]).wait()
        pltpu.make_async_copy(ab_hbm.at[pl.ds(0, P1_BE)], b_v.at[gs],
                              sems_b[gs]).wait()

    def compute_block(par, gs, eoff, ne, mbase):
        def edge_body(e, _):
            ve = esb[par][pl.ds(eoff + e * EDGE_D, 16)]
            for g in range(4):
                v = (a_v[gs, e, pl.ds(g * 16, 16)]
                     + b_v[gs, e, pl.ds(D + g * 16, 16)])
                for k in range(4):
                    v = v + ve[k] * w1c_vecs[k][g]
                v = jnp.maximum(v, 0.0)
                m_v[e, pl.ds(g * 16, 16)] = v
                sacc[g] = sacc[g] + v
                sacc[g + 4] = sacc[g + 4] + v * v
            return 0

        lax.fori_loop(0, ne, edge_body, 0)
        pltpu.sync_copy(m_v.at[pl.ds(0, ne)], m_hbm.at[pl.ds(mbase, ne)])

    start_sb(0, 0)
    start_sb(1, 1)

    def pair_body(j2, _):
        for par in (0, 1):
            jj = 2 * j2 + par
            pltpu.make_async_copy(dst_hbm.at[pl.ds(0, SB)], dsb[par],
                                  sems_i[par]).wait()
            pltpu.make_async_copy(src_hbm.at[pl.ds(0, SB)], ssb[par],
                                  sems_i[par]).wait()
            pltpu.make_async_copy(ea_hbm.at[pl.ds(0, SB * EDGE_D)],
                                  esb[par].at[pl.ds(0, SB * EDGE_D)],
                                  sems_e[par]).wait()
            start_gather(par, 0, 0)
            for k in range(SB_NB):
                gs = k % 2
                wait_gather(gs)
                if k + 1 < SB_NB:
                    start_gather(par, k + 1, (k + 1) % 2)
                compute_block(par, gs, k * P1_BE * EDGE_D, P1_BE,
                              base0 + jj * SB + k * P1_BE)

            @pl.when(jj + 2 < NSB)
            def _():
                start_sb(jj + 2, par)
        return 0

    lax.fori_loop(0, NSB // 2, pair_body, 0)

    tb = base0 + NSB * SB
    pltpu.sync_copy(dst_hbm.at[pl.ds(tb, P1_TB)], dsb0.at[pl.ds(0, P1_TB)])
    pltpu.sync_copy(src_hbm.at[pl.ds(tb, P1_TB)], ssb0.at[pl.ds(0, P1_TB)])
    pltpu.sync_copy(ea_hbm.at[pl.ds(tb * EDGE_D, P1_TB * EDGE_D)],
                    esb0.at[pl.ds(0, P1_TB * EDGE_D)])
    for k in range(P1_TNB + 1):
        ne = P1_BE if k < P1_TNB else P1_TAIL
        pltpu.async_copy(ab_hbm.at[dsb0.at[pl.ds(k * P1_BE, ne)]],
                         a_v.at[0, pl.ds(0, ne)], sem_a0).wait()
        pltpu.async_copy(ab_hbm.at[ssb0.at[pl.ds(k * P1_BE, ne)]],
                         b_v.at[0, pl.ds(0, ne)], sem_b0).wait()
        compute_block(0, 0, k * P1_BE * EDGE_D, ne, tb + k * P1_BE)

    for g in range(4):
        stats_v[0, pl.ds(g * 16, 16)] = sacc[g]
        stats_v[1, pl.ds(g * 16, 16)] = sacc[g + 4]
    pltpu.sync_copy(stats_v, stats_hbm.at[wid])


_msg_stats = pl.kernel(
    _msg_stats_body,
    out_type=[jax.ShapeDtypeStruct((E, D), jnp.float32),
              jax.ShapeDtypeStruct((NW, 2, D), jnp.float32)],
    mesh=_mesh,
    scratch_types=[pltpu.VMEM((SB,), jnp.int32),
                   pltpu.VMEM((SB,), jnp.int32),
                   pltpu.VMEM((SB,), jnp.int32),
                   pltpu.VMEM((SB,), jnp.int32),
                   pltpu.VMEM((SB * EDGE_D + 16,), jnp.float32),
                   pltpu.VMEM((SB * EDGE_D + 16,), jnp.float32),
                   pltpu.VMEM((2, P1_BE, 2 * D), jnp.float32),
                   pltpu.VMEM((2, P1_BE, 2 * D), jnp.float32),
                   pltpu.VMEM((P1_BE, D), jnp.float32),
                   pltpu.VMEM((EDGE_D, D), jnp.float32),
                   pltpu.VMEM((8, 16), jnp.float32),
                   pltpu.VMEM((2, D), jnp.float32),
                   pltpu.SemaphoreType.DMA, pltpu.SemaphoreType.DMA,
                   pltpu.SemaphoreType.DMA, pltpu.SemaphoreType.DMA,
                   pltpu.SemaphoreType.DMA, pltpu.SemaphoreType.DMA,
                   pltpu.SemaphoreType.DMA, pltpu.SemaphoreType.DMA],
    name="msg_stats",
)


def _scatter_body(m2_hbm, dst_hbm, aggr_hbm,
                  didx, didx_t, cidx, mb_v, zb_v, st_v, acc):
    c = lax.axis_index("c")
    s = lax.axis_index("s")

    def zb_body(i, _):
        zb_v[i, pl.ds(0, 16)] = jnp.zeros((16,), jnp.float32)
        zb_v[i, pl.ds(16, 16)] = jnp.zeros((16,), jnp.float32)
        return 0

    lax.fori_loop(0, ZCH, zb_body, 0)
    lanes = lax.iota(jnp.int32, 16)

    def zcp_body(j, _):
        cidx[...] = s * NPS + j * ZCH + lanes
        pltpu.sync_copy(zb_v, acc.at[cidx])
        return 0

    lax.fori_loop(0, NPS // ZCH, zcp_body, 0)
    plsc.subcore_barrier()

    base0 = s * EPS_SC

    def do_blk(idx_ref, base, ne):
        pltpu.sync_copy(dst_hbm.at[pl.ds(base, ne)], idx_ref)
        pltpu.sync_copy(m2_hbm.at[c, pl.ds(base, ne)], mb_v.at[pl.ds(0, ne)])
        pltpu.sync_copy(mb_v.at[pl.ds(0, ne)], acc.at[idx_ref], add=True)

    def blk_body(i, _):
        do_blk(didx, base0 + i * SC_BE, SC_BE)
        return 0

    lax.fori_loop(0, SC_NB, blk_body, 0)
    do_blk(didx_t, base0 + SC_NB * SC_BE, SC_TAIL)
    plsc.subcore_barrier()

    def wcp_body(j, _):
        cidx[...] = s * NPS + j * ZCH + lanes
        pltpu.sync_copy(acc.at[cidx], st_v)
        pltpu.sync_copy(st_v, aggr_hbm.at[c, pl.ds(s * NPS + j * ZCH, ZCH)])
        return 0

    lax.fori_loop(0, NPS // ZCH, wcp_body, 0)


_scatter = pl.kernel(
    _scatter_body,
    out_type=jax.ShapeDtypeStruct((NC, NPAD, DH), jnp.float32),
    mesh=_mesh,
    scratch_types=[pltpu.VMEM((SC_BE,), jnp.int32),
                   pltpu.VMEM((SC_TAIL,), jnp.int32),
                   pltpu.VMEM((ZCH,), jnp.int32),
                   pltpu.VMEM((SC_BE, DH), jnp.float32),
                   pltpu.VMEM((ZCH, DH), jnp.float32),
                   pltpu.VMEM((ZCH, DH), jnp.float32),
                   pltpu.VMEM_SHARED((ACC_R, DH), jnp.float32)],
    compiler_params=pltpu.CompilerParams(use_tc_tiling_on_sc=False),
    name="edge_scatter",
)


def _dot(a, b):
    return jnp.dot(a, b, preferred_element_type=jnp.float32)


def _embed_body(xp_ref, w_ref, b_ref, wa_ref, wb_ref, b1_ref,
                h_ref, ab_ref):
    h = _dot(xp_ref[...], w_ref[...]) + b_ref[...]
    h_ref[...] = h
    ab_ref[...] = jnp.concatenate(
        [_dot(h, wa_ref[...]) + b1_ref[...], _dot(h, wb_ref[...])], axis=1)


def _embed_call(xp, w, b, wa, wb, b1):
    row = lambda i: (i, 0)
    return pl.pallas_call(
        _embed_body,
        grid=(N_NB,),
        in_specs=[pl.BlockSpec((NBLK, ATOM3), row),
                  pl.BlockSpec((ATOM3, D), lambda i: (0, 0)),
                  pl.BlockSpec((1, D), lambda i: (0, 0)),
                  pl.BlockSpec((D, D), lambda i: (0, 0)),
                  pl.BlockSpec((D, D), lambda i: (0, 0)),
                  pl.BlockSpec((1, D), lambda i: (0, 0))],
        out_specs=[pl.BlockSpec((NBLK, D), row),
                   pl.BlockSpec((NBLK, 2 * D), row)],
        out_shape=[jax.ShapeDtypeStruct((N, D), jnp.float32),
                   jax.ShapeDtypeStruct((N, 2 * D), jnp.float32)],
    )(xp, w, b, wa, wb, b1)


ATOM3 = 14


def _mm_body(m_ref, w_ref, b_ref, o_ref):
    m2 = jnp.maximum(_dot(m_ref[...], w_ref[...]) + b_ref[...], 0.0)
    o_ref[0] = m2[:, :DH]
    o_ref[1] = m2[:, DH:]


def _mm_call(m, w2eff, b2eff):
    return pl.pallas_call(
        _mm_body,
        grid=(M_NB,),
        in_specs=[pl.BlockSpec((MBLK, D), lambda i: (i, 0)),
                  pl.BlockSpec((D, D), lambda i: (0, 0)),
                  pl.BlockSpec((1, D), lambda i: (0, 0))],
        out_specs=pl.BlockSpec((NC, MBLK, DH), lambda i: (0, i, 0)),
        out_shape=jax.ShapeDtypeStruct((NC, E, DH), jnp.float32),
    )(m, w2eff, b2eff)


def _upd1_body(h_ref, ag_ref, ua_ref, ub_ref, b1_ref, u1_ref, st_ref):
    i = pl.program_id(0)
    ag = jnp.concatenate([ag_ref[0], ag_ref[1]], axis=1)
    u1 = jnp.maximum(_dot(h_ref[...], ua_ref[...])
                     + _dot(ag, ub_ref[...]) + b1_ref[...], 0.0)
    u1_ref[...] = u1
    ps = jnp.sum(u1, axis=0, keepdims=True)
    pq = jnp.sum(u1 * u1, axis=0, keepdims=True)
    blk = jnp.concatenate([ps, pq], axis=0)

    @pl.when(i == 0)
    def _():
        st_ref[...] = blk

    @pl.when(i > 0)
    def _():
        st_ref[...] = st_ref[...] + blk


def _upd1_call(h, aggr, ua, ub, b1):
    return pl.pallas_call(
        _upd1_body,
        grid=(N_NB,),
        in_specs=[pl.BlockSpec((NBLK, D), lambda i: (i, 0)),
                  pl.BlockSpec((NC, NBLK, DH), lambda i: (0, i, 0)),
                  pl.BlockSpec((D, D), lambda i: (0, 0)),
                  pl.BlockSpec((D, D), lambda i: (0, 0)),
                  pl.BlockSpec((1, D), lambda i: (0, 0))],
        out_specs=[pl.BlockSpec((NBLK, D), lambda i: (i, 0)),
                   pl.BlockSpec((2, D), lambda i: (0, 0))],
        out_shape=[jax.ShapeDtypeStruct((N, D), jnp.float32),
                   jax.ShapeDtypeStruct((2, D), jnp.float32)],
    )(h, aggr, ua, ub, b1)


def _upd2ab_body(h_ref, u1_ref, u2_ref, b2_ref, wa_ref, wb_ref, b1n_ref,
                 hn_ref, ab_ref):
    u = jnp.maximum(_dot(u1_ref[...], u2_ref[...]) + b2_ref[...], 0.0)
    hn = h_ref[...] + u
    hn_ref[...] = hn
    ab_ref[...] = jnp.concatenate(
        [_dot(hn, wa_ref[...]) + b1n_ref[...], _dot(hn, wb_ref[...])], axis=1)


def _upd2ab_call(h, u1, u2eff, ub2eff, wa, wb, b1n):
    return pl.pallas_call(
        _upd2ab_body,
        grid=(N_NB,),
        in_specs=[pl.BlockSpec((NBLK, D), lambda i: (i, 0)),
                  pl.BlockSpec((NBLK, D), lambda i: (i, 0)),
                  pl.BlockSpec((D, D), lambda i: (0, 0)),
                  pl.BlockSpec((1, D), lambda i: (0, 0)),
                  pl.BlockSpec((D, D), lambda i: (0, 0)),
                  pl.BlockSpec((D, D), lambda i: (0, 0)),
                  pl.BlockSpec((1, D), lambda i: (0, 0))],
        out_specs=[pl.BlockSpec((NBLK, D), lambda i: (i, 0)),
                   pl.BlockSpec((NBLK, 2 * D), lambda i: (i, 0))],
        out_shape=[jax.ShapeDtypeStruct((N, D), jnp.float32),
                   jax.ShapeDtypeStruct((N, 2 * D), jnp.float32)],
    )(h, u1, u2eff, ub2eff, wa, wb, b1n)


def _upd2_body(h_ref, u1_ref, u2_ref, b2_ref, hn_ref):
    u = jnp.maximum(_dot(u1_ref[...], u2_ref[...]) + b2_ref[...], 0.0)
    hn_ref[...] = h_ref[...] + u


def _upd2_call(h, u1, u2eff, ub2eff):
    return pl.pallas_call(
        _upd2_body,
        grid=(N_NB,),
        in_specs=[pl.BlockSpec((NBLK, D), lambda i: (i, 0)),
                  pl.BlockSpec((NBLK, D), lambda i: (i, 0)),
                  pl.BlockSpec((D, D), lambda i: (0, 0)),
                  pl.BlockSpec((1, D), lambda i: (0, 0))],
        out_specs=pl.BlockSpec((NBLK, D), lambda i: (i, 0)),
        out_shape=jax.ShapeDtypeStruct((N, D), jnp.float32),
    )(h, u1, u2eff, ub2eff)


def _pool_body(b_ref, h_ref, pw_ref, pb_ref, out_ref, sums, cnts):
    i = pl.program_id(0)

    @pl.when(i == 0)
    def _():
        sums[...] = jnp.zeros_like(sums)
        cnts[...] = jnp.zeros_like(cnts)

    brow = b_ref[0]
    gids = lax.broadcasted_iota(jnp.int32, (G, NBLK), 0)
    onehot = (gids == brow).astype(jnp.float32)
    sums[...] += _dot(onehot, h_ref[...])
    cnts[...] += jnp.sum(onehot, axis=1, keepdims=True)

    @pl.when(i == N_NB - 1)
    def _():
        hg = sums[...] / jnp.maximum(cnts[...], 1.0)
        out_ref[...] = _dot(hg, pw_ref[...]) + pb_ref[...]


def _pool_call(batch3, h, pw, pb):
    return pl.pallas_call(
        _pool_body,
        grid=(N_NB,),
        in_specs=[pl.BlockSpec((1, 1, NBLK), lambda i: (i, 0, 0)),
                  pl.BlockSpec((NBLK, D), lambda i: (i, 0)),
                  pl.BlockSpec((D, 1), lambda i: (0, 0)),
                  pl.BlockSpec((1, 1), lambda i: (0, 0))],
        out_specs=pl.BlockSpec((G, 1), lambda i: (0, 0)),
        out_shape=jax.ShapeDtypeStruct((G, 1), jnp.float32),
        scratch_shapes=[pltpu.VMEM((G, D), jnp.float32),
                        pltpu.VMEM((G, 1), jnp.float32)],
    )(batch3, h, pw, pb)


def kernel(x, pos, edge_index, edge_attr, batch, lin_in_W, lin_in_b,
           msg_W1, msg_b1, msg_g, msg_beta, msg_W2, msg_b2,
           upd_W1, upd_b1, upd_g, upd_beta, upd_W2, upd_b2,
           pred_W, pred_b):
    src = edge_index[0]
    dst = edge_index[1]
    eaf = edge_attr.reshape(E * EDGE_D)
    xp = jnp.concatenate([x, pos], axis=1)
    h, AB = _embed_call(xp, lin_in_W, lin_in_b.reshape(1, D),
                        msg_W1[0, :D], msg_W1[0, D:2 * D],
                        msg_b1[0].reshape(1, D))
    for l in range(L):
        w1c = msg_W1[l, 2 * D:]
        m, pstats = _msg_stats(AB, dst, src, eaf, w1c)
        st = jnp.sum(pstats, axis=0)
        mu = st[0] / E
        var = st[1] / E - mu * mu
        sg = msg_g[l] * lax.rsqrt(var + EPS)
        t = msg_beta[l] - mu * sg
        w2eff = sg[:, None] * msg_W2[l]
        b2eff = t @ msg_W2[l] + msg_b2[l]
        m2 = _mm_call(m, w2eff, b2eff.reshape(1, D))
        aggr = _scatter(m2, dst)
        u1, st2 = _upd1_call(h, aggr, upd_W1[l, :D], upd_W1[l, D:],
                             upd_b1[l].reshape(1, D))
        mu2 = st2[0] / N
        var2 = st2[1] / N - mu2 * mu2
        sg2 = upd_g[l] * lax.rsqrt(var2 + EPS)
        t2 = upd_beta[l] - mu2 * sg2
        u2eff = sg2[:, None] * upd_W2[l]
        ub2eff = t2 @ upd_W2[l] + upd_b2[l]
        if l < L - 1:
            h, AB = _upd2ab_call(h, u1, u2eff, ub2eff.reshape(1, D),
                                 msg_W1[l + 1, :D], msg_W1[l + 1, D:2 * D],
                                 msg_b1[l + 1].reshape(1, D))
        else:
            h = _upd2_call(h, u1, u2eff, ub2eff.reshape(1, D))
    out = _pool_call(batch.reshape(N_NB, 1, NBLK), h, pred_W,
                     pred_b.reshape(1, 1))
    return out.reshape(-1)
